# baseline (speedup 1.0000x reference)
import jax
import jax.numpy as jnp
from jax import lax
from jax.experimental import pallas as pl
from jax.experimental.pallas import tpu as pltpu

N = 100000
H = 512
NUM_GRAPHS = 256
BN = 4000
NBLK = N // BN


def _stage1(v_ref, w1_ref, b1_ref, batch_ref, out_ref):
    vb = v_ref[...].astype(jnp.bfloat16)
    h = lax.dot_general(vb, w1_ref[...], (((1,), (1,)), ((), ())),
                        preferred_element_type=jnp.float32)
    h = jnp.maximum(h + b1_ref[...], 0.0)

    ids = batch_ref[0, 0, :]
    seg = lax.broadcasted_iota(jnp.int32, (NUM_GRAPHS, BN), 0)
    onehot = (seg == ids[None, :]).astype(jnp.bfloat16)
    part = lax.dot_general(onehot, h.astype(jnp.bfloat16),
                           (((1,), (0,)), ((), ())),
                           preferred_element_type=jnp.float32)
    out_ref[0] = part


def _stage2(parts_ref, w2_ref, b2_ref, out_ref):
    pooled = jnp.sum(parts_ref[...], axis=0)
    o = lax.dot_general(pooled, w2_ref[...], (((1,), (1,)), ((), ())),
                        preferred_element_type=jnp.float32)
    out_ref[...] = jnp.maximum(o + b2_ref[...], 0.0)


@jax.jit
def kernel(v, W1, b1, W2, b2, batch):
    batch32 = batch.astype(jnp.int32).reshape(NBLK, 1, BN)
    W1 = W1.astype(jnp.bfloat16)
    b1r = b1.reshape(1, H)
    b2r = b2.reshape(1, H)
    parts = pl.pallas_call(
        _stage1,
        grid=(NBLK,),
        in_specs=[
            pl.BlockSpec((BN, 3 * H), lambda i: (i, 0)),
            pl.BlockSpec((H, 3 * H), lambda i: (0, 0)),
            pl.BlockSpec((1, H), lambda i: (0, 0)),
            pl.BlockSpec((1, 1, BN), lambda i: (i, 0, 0)),
        ],
        out_specs=pl.BlockSpec((1, NUM_GRAPHS, H), lambda i: (i, 0, 0)),
        out_shape=jax.ShapeDtypeStruct((NBLK, NUM_GRAPHS, H), jnp.float32),
        compiler_params=pltpu.CompilerParams(
            dimension_semantics=("parallel",)),
    )(v, W1, b1r, batch32)
    out = pl.pallas_call(
        _stage2,
        in_specs=[
            pl.BlockSpec((NBLK, NUM_GRAPHS, H), lambda: (0, 0, 0)),
            pl.BlockSpec((H, H), lambda: (0, 0)),
            pl.BlockSpec((1, H), lambda: (0, 0)),
        ],
        out_specs=pl.BlockSpec((NUM_GRAPHS, H), lambda: (0, 0)),
        out_shape=jax.ShapeDtypeStruct((NUM_GRAPHS, H), jnp.float32),
    )(parts, W2, b2r)
    return out


# speedup vs baseline: 1.0060x; 1.0060x over previous
import functools

import jax
import jax.numpy as jnp
from jax import lax
from jax.experimental import pallas as pl
from jax.experimental.pallas import tpu as pltpu

N = 100000
H = 512
NUM_GRAPHS = 256
BN = 2000
NBLK = N // (2 * BN)


def _half(v_ref, batch_ref, w1_ref, b1_ref, acc_ref):
    vb = v_ref[...].astype(jnp.bfloat16)
    h = lax.dot_general(vb, w1_ref[...], (((1,), (1,)), ((), ())),
                        preferred_element_type=jnp.float32)
    h = jnp.maximum(h + b1_ref[...], 0.0)
    ids = batch_ref[0, 0, :]
    seg = lax.broadcasted_iota(jnp.int32, (NUM_GRAPHS, BN), 0)
    onehot = (seg == ids[None, :]).astype(jnp.bfloat16)
    part = lax.dot_general(onehot, h.astype(jnp.bfloat16),
                           (((1,), (0,)), ((), ())),
                           preferred_element_type=jnp.float32)
    acc_ref[...] += part


def _fused_kernel(va_ref, vb_ref, w1_ref, b1_ref, w2_ref, b2_ref,
                  batcha_ref, batchb_ref, out_ref, acc_ref):
    i = pl.program_id(0)

    @pl.when(i == 0)
    def _init():
        acc_ref[...] = jnp.zeros_like(acc_ref)

    _half(va_ref, batcha_ref, w1_ref, b1_ref, acc_ref)
    _half(vb_ref, batchb_ref, w1_ref, b1_ref, acc_ref)

    @pl.when(i == NBLK - 1)
    def _fin():
        pooled = acc_ref[...]
        o = lax.dot_general(pooled, w2_ref[...], (((1,), (1,)), ((), ())),
                            preferred_element_type=jnp.float32)
        out_ref[...] = jnp.maximum(o + b2_ref[...], 0.0)


@functools.partial(jax.jit, static_argnames=())
def kernel(v, W1, b1, W2, b2, batch):
    batch32 = batch.astype(jnp.int32).reshape(2 * NBLK, 1, BN)
    W1 = W1.astype(jnp.bfloat16)
    b1r = b1.reshape(1, H)
    b2r = b2.reshape(1, H)
    out = pl.pallas_call(
        _fused_kernel,
        grid=(NBLK,),
        in_specs=[
            pl.BlockSpec((BN, 3 * H), lambda i: (2 * i, 0)),
            pl.BlockSpec((BN, 3 * H), lambda i: (2 * i + 1, 0)),
            pl.BlockSpec((H, 3 * H), lambda i: (0, 0)),
            pl.BlockSpec((1, H), lambda i: (0, 0)),
            pl.BlockSpec((H, H), lambda i: (0, 0)),
            pl.BlockSpec((1, H), lambda i: (0, 0)),
            pl.BlockSpec((1, 1, BN), lambda i: (2 * i, 0, 0)),
            pl.BlockSpec((1, 1, BN), lambda i: (2 * i + 1, 0, 0)),
        ],
        out_specs=pl.BlockSpec((NUM_GRAPHS, H), lambda i: (0, 0)),
        out_shape=jax.ShapeDtypeStruct((NUM_GRAPHS, H), jnp.float32),
        scratch_shapes=[pltpu.VMEM((NUM_GRAPHS, H), jnp.float32)],
    )(v, v, W1, b1r, W2, b2r, batch32, batch32)
    return out


# speedup vs baseline: 1.0441x; 1.0378x over previous
import functools

import jax
import jax.numpy as jnp
from jax import lax
from jax.experimental import pallas as pl
from jax.experimental.pallas import tpu as pltpu

N = 100000
H = 512
NUM_GRAPHS = 256
BN = 4000
NBLK = N // BN


def _fused_kernel(v_ref, w1_ref, b1_ref, w2_ref, b2_ref, batch_ref, out_ref,
                  acc_ref):
    i = pl.program_id(0)

    @pl.when(i == 0)
    def _init():
        acc_ref[...] = jnp.zeros_like(acc_ref)

    vb = v_ref[...].astype(jnp.bfloat16)
    h = lax.dot_general(vb, w1_ref[...], (((1,), (1,)), ((), ())),
                        preferred_element_type=jnp.float32)
    h = jnp.maximum(h + b1_ref[...], 0.0)

    ids = batch_ref[0, 0, :]
    seg = lax.broadcasted_iota(jnp.int32, (NUM_GRAPHS, BN), 0)
    onehot = (seg == ids[None, :]).astype(jnp.float32)
    part = lax.dot_general(onehot, h, (((1,), (0,)), ((), ())),
                           preferred_element_type=jnp.float32)
    acc_ref[...] += part

    @pl.when(i == NBLK - 1)
    def _fin():
        pooled = acc_ref[...]
        o = lax.dot_general(pooled, w2_ref[...], (((1,), (1,)), ((), ())),
                            preferred_element_type=jnp.float32)
        out_ref[...] = jnp.maximum(o + b2_ref[...], 0.0)


@functools.partial(jax.jit, static_argnames=())
def kernel(v, W1, b1, W2, b2, batch):
    batch32 = batch.astype(jnp.int32).reshape(NBLK, 1, BN)
    W1 = W1.astype(jnp.bfloat16)
    b1r = b1.reshape(1, H)
    b2r = b2.reshape(1, H)
    out = pl.pallas_call(
        _fused_kernel,
        grid=(NBLK,),
        in_specs=[
            pl.BlockSpec((BN, 3 * H), lambda i: (i, 0)),
            pl.BlockSpec((H, 3 * H), lambda i: (0, 0)),
            pl.BlockSpec((1, H), lambda i: (0, 0)),
            pl.BlockSpec((H, H), lambda i: (0, 0)),
            pl.BlockSpec((1, H), lambda i: (0, 0)),
            pl.BlockSpec((1, 1, BN), lambda i: (i, 0, 0)),
        ],
        out_specs=pl.BlockSpec((NUM_GRAPHS, H), lambda i: (0, 0)),
        out_shape=jax.ShapeDtypeStruct((NUM_GRAPHS, H), jnp.float32),
        scratch_shapes=[pltpu.VMEM((NUM_GRAPHS, H), jnp.float32)],
    )(v, W1, b1r, W2, b2r, batch32)
    return out
